# baseline (speedup 1.0000x reference)
import functools

import jax
import jax.numpy as jnp
from jax import lax
from jax.experimental import pallas as pl
from jax.experimental.pallas import tpu as pltpu
from jax.experimental.pallas import tpu_sc as plsc

_G = 200
_NBINS = _G * _G
_N = 25000 * 81
_NPAD = 2 ** 21
_NC = 2
_NS = 16
_NW = _NC * _NS
_PPW = _NPAD // _NW
_CH = 16384
_KCH = _PPW // _CH
_NGRP = _CH // 16
_CLIP_HI = _G - 1 - 1e-6


def _sc_hist_body(xs_hbm, ys_hbm, out_hbm, xb0, yb0, xb1, yb1, hist, sems):
  xbufs = (xb0, xb1)
  ybufs = (yb0, yb1)
  c = lax.axis_index("c")
  s = lax.axis_index("s")
  wid = c * _NS + s
  base = wid * _PPW

  zeros16 = jnp.zeros((16,), jnp.float32)

  @pl.loop(0, _NBINS // 16, unroll=8)
  def _(i):
    hist[pl.ds(i * 16, 16)] = zeros16

  ones16 = jnp.ones((16,), jnp.float32)
  iota = lax.iota(jnp.int32, 16)

  def start_dma(k, b):
    off = base + k * _CH
    pltpu.async_copy(xs_hbm.at[pl.ds(off, _CH)], xbufs[b], sems.at[b])
    pltpu.async_copy(ys_hbm.at[pl.ds(off, _CH)], ybufs[b], sems.at[b])

  def wait_dma(b):
    pltpu.make_async_copy(
        xs_hbm.at[pl.ds(0, _CH)], xbufs[b], sems.at[b]
    ).wait()
    pltpu.make_async_copy(
        ys_hbm.at[pl.ds(0, _CH)], ybufs[b], sems.at[b]
    ).wait()

  start_dma(0, 0)
  start_dma(1, 1)

  def bin_index(xbuf, ybuf, g):
    g16 = g * 16
    xv = xbuf[pl.ds(g16, 16)]
    yv = ybuf[pl.ds(g16, 16)]
    xc = jnp.clip(xv, 0.0, _CLIP_HI)
    yc = jnp.clip(yv, 0.0, _CLIP_HI)
    xi = (xc + 0.5).astype(jnp.int32)
    yi = (yc + 0.5).astype(jnp.int32)
    return xi * _G + yi

  def process_chunk(k, b):
    wait_dma(b)
    xbuf = xbufs[b]
    ybuf = ybufs[b]
    thr = _N - (base + k * _CH)

    @pl.when(thr >= _CH)
    def _():
      @plsc.parallel_loop(0, _NGRP, unroll=16)
      def _(g):
        idx = bin_index(xbuf, ybuf, g)
        plsc.addupdate_scatter(hist, [idx], ones16)

    @pl.when(thr < _CH)
    def _():
      @plsc.parallel_loop(0, _NGRP, unroll=16)
      def _(g):
        idx = bin_index(xbuf, ybuf, g)
        m = (iota + g * 16) < thr
        plsc.addupdate_scatter(hist, [idx], ones16, mask=m)

    @pl.when(k + 2 < _KCH)
    def _():
      start_dma(k + 2, b)

  @pl.loop(0, _KCH, step=2)
  def _(k0):
    process_chunk(k0, 0)
    process_chunk(k0 + 1, 1)

  pltpu.sync_copy(hist, out_hbm.at[wid])


_sc_hist = pl.kernel(
    _sc_hist_body,
    out_type=jax.ShapeDtypeStruct((_NW, _NBINS), jnp.float32),
    mesh=plsc.VectorSubcoreMesh(core_axis_name="c", subcore_axis_name="s"),
    scratch_types=[
        pltpu.VMEM((_CH,), jnp.float32),
        pltpu.VMEM((_CH,), jnp.float32),
        pltpu.VMEM((_CH,), jnp.float32),
        pltpu.VMEM((_CH,), jnp.float32),
        pltpu.VMEM((_NBINS,), jnp.float32),
        pltpu.SemaphoreType.DMA((2,)),
    ],
    compiler_params=pltpu.CompilerParams(needs_layout_passes=False),
)


def _finalize_body(partials_ref, grid_ref, out_ref):
  total = jnp.sum(partials_ref[...], axis=0)
  prob = total / float(25000 * 80)
  out_ref[...] = jnp.where(grid_ref[...] != 0.0, 0.0, prob)


def kernel(all_points, grid):
  pts_t = all_points.T
  zpad = jnp.zeros((_NPAD - _N,), jnp.float32)
  xs = jnp.concatenate([pts_t[0], zpad])
  ys = jnp.concatenate([pts_t[1], zpad])
  partials = _sc_hist(xs, ys)
  partials_3d = partials.reshape(_NW, _G, _G)
  out = pl.pallas_call(
      _finalize_body,
      out_shape=jax.ShapeDtypeStruct((_G, _G), jnp.float32),
  )(partials_3d, grid)
  return out


# speedup vs baseline: 1.6775x; 1.6775x over previous
import functools

import jax
import jax.numpy as jnp
from jax import lax
from jax.experimental import pallas as pl
from jax.experimental.pallas import tpu as pltpu
from jax.experimental.pallas import tpu_sc as plsc

_G = 200
_NBINS = _G * _G
_N = 25000 * 81
_NPAD = 2 ** 21
_NC = 2
_NS = 16
_NW = _NC * _NS
_PPW = _NPAD // _NW
_CH = 8192
_KCH = _PPW // _CH
_NGRP = _CH // 16
_CLIP_HI = _G - 1 - 1e-6


def _sc_hist_body(xs_hbm, ys_hbm, out_hbm, xb0, yb0, xb1, yb1, hist, sems):
  xbufs = (xb0, xb1)
  ybufs = (yb0, yb1)
  c = lax.axis_index("c")
  s = lax.axis_index("s")
  wid = c * _NS + s
  base = wid * _PPW

  zeros16 = jnp.zeros((16,), jnp.float32)

  @pl.loop(0, _NBINS // 16, unroll=8)
  def _(i):
    hist[pl.ds(i * 16, 16)] = zeros16

  ones16 = jnp.ones((16,), jnp.float32)
  iota = lax.iota(jnp.int32, 16)

  def start_dma(k, b):
    off = base + k * _CH
    pltpu.async_copy(xs_hbm.at[pl.ds(off, _CH)], xbufs[b], sems.at[b])
    pltpu.async_copy(ys_hbm.at[pl.ds(off, _CH)], ybufs[b], sems.at[b])

  def wait_dma(b):
    pltpu.make_async_copy(
        xs_hbm.at[pl.ds(0, _CH)], xbufs[b], sems.at[b]
    ).wait()
    pltpu.make_async_copy(
        ys_hbm.at[pl.ds(0, _CH)], ybufs[b], sems.at[b]
    ).wait()

  start_dma(0, 0)
  start_dma(1, 1)

  def bin_index(xbuf, ybuf, g):
    g16 = g * 16
    xv = xbuf[pl.ds(g16, 16)]
    yv = ybuf[pl.ds(g16, 16)]
    xc = jnp.clip(xv, 0.0, _CLIP_HI)
    yc = jnp.clip(yv, 0.0, _CLIP_HI)
    xi = (xc + 0.5).astype(jnp.int32)
    yi = (yc + 0.5).astype(jnp.int32)
    return xi * _G + yi

  def process_chunk(k, b):
    wait_dma(b)
    xbuf = xbufs[b]
    ybuf = ybufs[b]
    thr = _N - (base + k * _CH)

    @pl.when(thr >= _CH)
    def _():
      @plsc.parallel_loop(0, _NGRP, unroll=8)
      def _(g):
        idx = bin_index(xbuf, ybuf, g)
        plsc.addupdate_scatter(hist, [idx], ones16)

    @pl.when(thr < _CH)
    def _():
      @plsc.parallel_loop(0, _NGRP, unroll=8)
      def _(g):
        idx = bin_index(xbuf, ybuf, g)
        m = (iota + g * 16) < thr
        plsc.addupdate_scatter(hist, [idx], ones16, mask=m)

    @pl.when(k + 2 < _KCH)
    def _():
      start_dma(k + 2, b)

  @pl.loop(0, _KCH, step=2)
  def _(k0):
    process_chunk(k0, 0)
    process_chunk(k0 + 1, 1)

  pltpu.sync_copy(hist, out_hbm.at[wid])


_sc_hist = pl.kernel(
    _sc_hist_body,
    out_type=jax.ShapeDtypeStruct((_NW, _NBINS), jnp.float32),
    mesh=plsc.VectorSubcoreMesh(core_axis_name="c", subcore_axis_name="s"),
    scratch_types=[
        pltpu.VMEM((_CH,), jnp.float32),
        pltpu.VMEM((_CH,), jnp.float32),
        pltpu.VMEM((_CH,), jnp.float32),
        pltpu.VMEM((_CH,), jnp.float32),
        pltpu.VMEM((_NBINS,), jnp.float32),
        pltpu.SemaphoreType.DMA((2,)),
    ],
    compiler_params=pltpu.CompilerParams(needs_layout_passes=False),
)


def _finalize_body(partials_ref, grid_ref, out_ref):
  total = jnp.sum(partials_ref[...], axis=0)
  prob = total / float(25000 * 80)
  out_ref[...] = jnp.where(grid_ref[...] != 0.0, 0.0, prob)


def kernel(all_points, grid):
  padded = jnp.zeros((2, _NPAD), jnp.float32).at[:, :_N].set(all_points.T)
  partials = _sc_hist(padded[0], padded[1])
  grid_flat = grid.reshape(_NBINS)
  out_flat = pl.pallas_call(
      _finalize_body,
      out_shape=jax.ShapeDtypeStruct((_NBINS,), jnp.float32),
  )(partials, grid_flat)
  return out_flat.reshape(_G, _G)
